# baseline (speedup 1.0000x reference)
import jax
import jax.numpy as jnp
from jax.experimental import pallas as pl
from jax.experimental.pallas import tpu as pltpu

_V = 10000
_SMOOTHING = 0.1
_CONFIDENCE = 1.0 - _SMOOTHING
_SV = _SMOOTHING / (_V - 1)
_CMS = _CONFIDENCE - _SV

_TBLK = 64


def _loss_kernel(lens_ref, text_ref, train_ref, out_ref, carry_ref):
    b = pl.program_id(0)
    j = pl.program_id(1)
    L = lens_ref[b]
    t0 = j * _TBLK

    @pl.when((b == 0) & (j == 0))
    def _init():
        out_ref[0, 0] = 0.0

    @pl.when(t0 <= L + 1)
    def _text_work():
        tb = text_ref[0]
        tmax = jnp.max(tb, axis=-1, keepdims=True)
        iota_v = jax.lax.broadcasted_iota(jnp.int32, tb.shape, 1)
        idx = jnp.min(jnp.where(tb == tmax, iota_v, _V), axis=-1, keepdims=True)

        @pl.when(j > 0)
        def _consume_carry():
            prev = carry_ref[:, :]
            iota1 = jax.lax.broadcasted_iota(jnp.int32, (1, _V), 1)
            xtp = jnp.sum(jnp.where(iota1 == idx[0:1, :], prev, 0.0))
            lse_p = jnp.log(jnp.sum(jnp.exp(prev)))
            sx_p = jnp.sum(prev)
            out_ref[0, 0] += lse_p - _SV * sx_p - _CMS * xtp

        @pl.when(t0 <= L)
        def _train_work():
            xb = train_ref[0]
            lse = jnp.log(jnp.sum(jnp.exp(xb), axis=-1, keepdims=True))
            sx = jnp.sum(xb, axis=-1, keepdims=True)
            xt = jnp.sum(
                jnp.where(iota_v[:-1] == idx[1:], xb[:-1], 0.0),
                axis=-1,
                keepdims=True,
            )
            tg = t0 + jax.lax.broadcasted_iota(jnp.int32, (_TBLK, 1), 0)
            act = (tg <= L).astype(jnp.float32)
            row_loss = lse[:-1] - _SV * sx[:-1] - _CMS * xt
            out_ref[0, 0] += jnp.sum(row_loss * act[:-1])
            carry_ref[:, :] = xb[_TBLK - 1 : _TBLK, :]


def kernel(text, lengths, train_outputs):
    B, T1, V = text.shape
    T = T1 - 1
    nj = T // _TBLK
    lens = jnp.asarray(lengths, jnp.int32)

    grid_spec = pltpu.PrefetchScalarGridSpec(
        num_scalar_prefetch=1,
        grid=(B, nj),
        in_specs=[
            pl.BlockSpec(
                (1, _TBLK, V),
                lambda b, j, lens: (b, jnp.minimum(j, (lens[b] + 1) // _TBLK), 0),
            ),
            pl.BlockSpec(
                (1, _TBLK, V),
                lambda b, j, lens: (b, jnp.minimum(j, lens[b] // _TBLK), 0),
            ),
        ],
        out_specs=pl.BlockSpec(memory_space=pltpu.SMEM),
        scratch_shapes=[pltpu.VMEM((1, V), jnp.float32)],
    )

    total = pl.pallas_call(
        _loss_kernel,
        grid_spec=grid_spec,
        out_shape=jax.ShapeDtypeStruct((1, 1), jnp.float32),
        compiler_params=pltpu.CompilerParams(
            dimension_semantics=("arbitrary", "arbitrary"),
        ),
    )(lens, text, train_outputs)

    count = jnp.sum(lens + 1).astype(jnp.float32)
    return total[0, 0] / count


# speedup vs baseline: 1.0263x; 1.0263x over previous
import jax
import jax.numpy as jnp
from jax.experimental import pallas as pl
from jax.experimental.pallas import tpu as pltpu

_V = 10000
_SMOOTHING = 0.1
_CONFIDENCE = 1.0 - _SMOOTHING
_SV = _SMOOTHING / (_V - 1)
_CMS = _CONFIDENCE - _SV

_TBLK = 64


def _loss_kernel(lens_ref, text_ref, train_ref, out_ref, carry_ref):
    b = pl.program_id(0)
    j = pl.program_id(1)
    L = lens_ref[b]
    t0 = j * _TBLK

    @pl.when((b == 0) & (j == 0))
    def _init():
        out_ref[0, 0] = 0.0

    @pl.when(t0 <= L + 1)
    def _text_work():
        tb = text_ref[0]
        tmax = jnp.max(tb, axis=-1, keepdims=True)

        @pl.when(j > 0)
        def _consume_carry():
            prev = carry_ref[:, :]
            xtp = jnp.sum(jnp.where(tb[0:1] == tmax[0:1], prev, 0.0))
            lse_p = jnp.log(jnp.sum(jnp.exp(prev)))
            sx_p = jnp.sum(prev)
            out_ref[0, 0] += lse_p - _SV * sx_p - _CMS * xtp

        @pl.when(t0 <= L)
        def _train_work():
            xb = train_ref[0]
            lse = jnp.log(jnp.sum(jnp.exp(xb), axis=-1, keepdims=True))
            sx = jnp.sum(xb, axis=-1, keepdims=True)
            xt = jnp.sum(
                jnp.where(tb[1:] == tmax[1:], xb[:-1], 0.0),
                axis=-1,
                keepdims=True,
            )
            tg = t0 + jax.lax.broadcasted_iota(jnp.int32, (_TBLK, 1), 0)
            act = (tg <= L).astype(jnp.float32)
            row_loss = lse[:-1] - _SV * sx[:-1] - _CMS * xt
            out_ref[0, 0] += jnp.sum(row_loss * act[:-1])
            carry_ref[:, :] = xb[_TBLK - 1 : _TBLK, :]


def kernel(text, lengths, train_outputs):
    B, T1, V = text.shape
    T = T1 - 1
    nj = T // _TBLK
    lens = jnp.asarray(lengths, jnp.int32)

    grid_spec = pltpu.PrefetchScalarGridSpec(
        num_scalar_prefetch=1,
        grid=(B, nj),
        in_specs=[
            pl.BlockSpec(
                (1, _TBLK, V),
                lambda b, j, lens: (b, jnp.minimum(j, (lens[b] + 1) // _TBLK), 0),
            ),
            pl.BlockSpec(
                (1, _TBLK, V),
                lambda b, j, lens: (b, jnp.minimum(j, lens[b] // _TBLK), 0),
            ),
        ],
        out_specs=pl.BlockSpec(memory_space=pltpu.SMEM),
        scratch_shapes=[pltpu.VMEM((1, V), jnp.float32)],
    )

    total = pl.pallas_call(
        _loss_kernel,
        grid_spec=grid_spec,
        out_shape=jax.ShapeDtypeStruct((1, 1), jnp.float32),
        compiler_params=pltpu.CompilerParams(
            dimension_semantics=("arbitrary", "arbitrary"),
        ),
    )(lens, text, train_outputs)

    count = jnp.sum(lens + 1).astype(jnp.float32)
    return total[0, 0] / count


# speedup vs baseline: 1.0350x; 1.0084x over previous
import jax
import jax.numpy as jnp
from jax.experimental import pallas as pl
from jax.experimental.pallas import tpu as pltpu

_V = 10000
_SMOOTHING = 0.1
_CONFIDENCE = 1.0 - _SMOOTHING
_SV = _SMOOTHING / (_V - 1)
_CMS = _CONFIDENCE - _SV

_TBLK = 128


def _loss_kernel(lens_ref, text_ref, train_ref, out_ref, carry_ref):
    b = pl.program_id(0)
    j = pl.program_id(1)
    L = lens_ref[b]
    t0 = j * _TBLK

    @pl.when((b == 0) & (j == 0))
    def _init():
        out_ref[0, 0] = 0.0

    @pl.when(t0 <= L + 1)
    def _text_work():
        tb = text_ref[0]
        tmax = jnp.max(tb, axis=-1, keepdims=True)

        @pl.when(j > 0)
        def _consume_carry():
            prev = carry_ref[:, :]
            xtp = jnp.sum(jnp.where(tb[0:1] == tmax[0:1], prev, 0.0))
            lse_p = jnp.log(jnp.sum(jnp.exp(prev)))
            sx_p = jnp.sum(prev)
            out_ref[0, 0] += lse_p - _SV * sx_p - _CMS * xtp

        @pl.when(t0 <= L)
        def _train_work():
            xb = train_ref[0]
            lse = jnp.log(jnp.sum(jnp.exp(xb), axis=-1, keepdims=True))
            sx = jnp.sum(xb, axis=-1, keepdims=True)
            xt = jnp.sum(
                jnp.where(tb[1:] == tmax[1:], xb[:-1], 0.0),
                axis=-1,
                keepdims=True,
            )
            tg = t0 + jax.lax.broadcasted_iota(jnp.int32, (_TBLK, 1), 0)
            act = (tg <= L).astype(jnp.float32)
            row_loss = lse[:-1] - _SV * sx[:-1] - _CMS * xt
            out_ref[0, 0] += jnp.sum(row_loss * act[:-1])
            carry_ref[:, :] = xb[_TBLK - 1 : _TBLK, :]


def kernel(text, lengths, train_outputs):
    B, T1, V = text.shape
    T = T1 - 1
    nj = T // _TBLK
    lens = jnp.asarray(lengths, jnp.int32)

    grid_spec = pltpu.PrefetchScalarGridSpec(
        num_scalar_prefetch=1,
        grid=(B, nj),
        in_specs=[
            pl.BlockSpec(
                (1, _TBLK, V),
                lambda b, j, lens: (b, jnp.minimum(j, (lens[b] + 1) // _TBLK), 0),
            ),
            pl.BlockSpec(
                (1, _TBLK, V),
                lambda b, j, lens: (b, jnp.minimum(j, lens[b] // _TBLK), 0),
            ),
        ],
        out_specs=pl.BlockSpec(memory_space=pltpu.SMEM),
        scratch_shapes=[pltpu.VMEM((1, V), jnp.float32)],
    )

    total = pl.pallas_call(
        _loss_kernel,
        grid_spec=grid_spec,
        out_shape=jax.ShapeDtypeStruct((1, 1), jnp.float32),
        compiler_params=pltpu.CompilerParams(
            dimension_semantics=("arbitrary", "arbitrary"),
        ),
    )(lens, text, train_outputs)

    count = jnp.sum(lens + 1).astype(jnp.float32)
    return total[0, 0] / count
